# baseline (speedup 1.0000x reference)
import jax
import jax.numpy as jnp
import numpy as np
from jax.experimental import pallas as pl
from jax.experimental.pallas import tpu as pltpu

_B, _N, _D = 8, 2048, 256
_P = _D // 4
_TOPK = 8
_TR = 256


def _graph_kernel(lt_ref, a_full_ref, a_tile_ref, wq_ref, bq_ref, wk_ref,
                  bk_ref, out_ref, k_scratch):
    i = pl.program_id(1)

    @pl.when(i == 0)
    def _():
        a = a_full_ref[0]
        k = jax.lax.dot_general(a, wk_ref[...],
                                (((1,), (1,)), ((), ())),
                                preferred_element_type=jnp.float32)
        k_scratch[...] = k + bk_ref[...]

    temperature = jnp.clip(jnp.exp(lt_ref[0, 0]), 0.1, 10.0)
    inv_scale = 1.0 / (np.sqrt(float(_P)) * temperature)

    a_tile = a_tile_ref[0]
    q = jax.lax.dot_general(a_tile, wq_ref[...],
                            (((1,), (1,)), ((), ())),
                            preferred_element_type=jnp.float32)
    q = q + bq_ref[...]

    s = jax.lax.dot_general(q, k_scratch[...],
                            (((1,), (1,)), ((), ())),
                            preferred_element_type=jnp.float32)
    s = s * inv_scale

    m1 = jnp.max(s, axis=-1, keepdims=True)
    s_work = jnp.where(s == m1, -jnp.inf, s)
    for _ in range(_TOPK - 2):
        m = jnp.max(s_work, axis=-1, keepdims=True)
        s_work = jnp.where(s_work == m, -jnp.inf, s_work)
    thr = jnp.max(s_work, axis=-1, keepdims=True)

    keep = s >= thr
    e = jnp.where(keep, jnp.exp(s - m1), 0.0)
    denom = jnp.sum(e, axis=-1, keepdims=True)
    out_ref[0] = e / denom


def kernel(action_states, Wq, bq, Wk, bk, log_temperature):
    lt = jnp.reshape(log_temperature, (1, 1))
    bq2 = jnp.reshape(bq, (1, _P))
    bk2 = jnp.reshape(bk, (1, _P))

    return pl.pallas_call(
        _graph_kernel,
        grid=(_B, _N // _TR),
        in_specs=[
            pl.BlockSpec((1, 1), lambda b, i: (0, 0)),
            pl.BlockSpec((1, _N, _D), lambda b, i: (b, 0, 0)),
            pl.BlockSpec((1, _TR, _D), lambda b, i: (b, i, 0)),
            pl.BlockSpec((_P, _D), lambda b, i: (0, 0)),
            pl.BlockSpec((1, _P), lambda b, i: (0, 0)),
            pl.BlockSpec((_P, _D), lambda b, i: (0, 0)),
            pl.BlockSpec((1, _P), lambda b, i: (0, 0)),
        ],
        out_specs=pl.BlockSpec((1, _TR, _N), lambda b, i: (b, i, 0)),
        out_shape=jax.ShapeDtypeStruct((_B, _N, _N), jnp.float32),
        scratch_shapes=[pltpu.VMEM((_N, _P), jnp.float32)],
        compiler_params=pltpu.CompilerParams(
            dimension_semantics=("parallel", "arbitrary")),
    )(lt, action_states, action_states, Wq, bq2, Wk, bk2)


# speedup vs baseline: 1.1032x; 1.1032x over previous
import jax
import jax.numpy as jnp
import numpy as np
from jax.experimental import pallas as pl
from jax.experimental.pallas import tpu as pltpu

_B, _N, _D = 8, 2048, 256
_P = _D // 4
_TOPK = 8
_TR = 1024


def _graph_kernel(lt_ref, a_full_ref, a_tile_ref, wq_ref, bq_ref, wk_ref,
                  bk_ref, out_ref, k_scratch):
    i = pl.program_id(1)

    @pl.when(i == 0)
    def _():
        a = a_full_ref[0]
        k = jax.lax.dot_general(a, wk_ref[...],
                                (((1,), (1,)), ((), ())),
                                preferred_element_type=jnp.float32)
        k_scratch[...] = k + bk_ref[...]

    temperature = jnp.clip(jnp.exp(lt_ref[0, 0]), 0.1, 10.0)
    inv_scale = 1.0 / (np.sqrt(float(_P)) * temperature)

    a_tile = a_tile_ref[0]
    q = jax.lax.dot_general(a_tile, wq_ref[...],
                            (((1,), (1,)), ((), ())),
                            preferred_element_type=jnp.float32)
    q = q + bq_ref[...]

    s = jax.lax.dot_general(q, k_scratch[...],
                            (((1,), (1,)), ((), ())),
                            preferred_element_type=jnp.float32)
    s = s * inv_scale

    m1 = jnp.max(s, axis=-1, keepdims=True)
    s_work = jnp.where(s == m1, -jnp.inf, s)
    for _ in range(_TOPK - 2):
        m = jnp.max(s_work, axis=-1, keepdims=True)
        s_work = jnp.where(s_work == m, -jnp.inf, s_work)
    thr = jnp.max(s_work, axis=-1, keepdims=True)

    keep = s >= thr
    e = jnp.where(keep, jnp.exp(s - m1), 0.0)
    denom = jnp.sum(e, axis=-1, keepdims=True)
    out_ref[0] = e / denom


def kernel(action_states, Wq, bq, Wk, bk, log_temperature):
    lt = jnp.reshape(log_temperature, (1, 1))
    bq2 = jnp.reshape(bq, (1, _P))
    bk2 = jnp.reshape(bk, (1, _P))

    return pl.pallas_call(
        _graph_kernel,
        grid=(_B, _N // _TR),
        in_specs=[
            pl.BlockSpec((1, 1), lambda b, i: (0, 0)),
            pl.BlockSpec((1, _N, _D), lambda b, i: (b, 0, 0)),
            pl.BlockSpec((1, _TR, _D), lambda b, i: (b, i, 0)),
            pl.BlockSpec((_P, _D), lambda b, i: (0, 0)),
            pl.BlockSpec((1, _P), lambda b, i: (0, 0)),
            pl.BlockSpec((_P, _D), lambda b, i: (0, 0)),
            pl.BlockSpec((1, _P), lambda b, i: (0, 0)),
        ],
        out_specs=pl.BlockSpec((1, _TR, _N), lambda b, i: (b, i, 0)),
        out_shape=jax.ShapeDtypeStruct((_B, _N, _N), jnp.float32),
        scratch_shapes=[pltpu.VMEM((_N, _P), jnp.float32)],
        compiler_params=pltpu.CompilerParams(
            dimension_semantics=("parallel", "arbitrary")),
    )(lt, action_states, action_states, Wq, bq2, Wk, bk2)


# speedup vs baseline: 1.6823x; 1.5249x over previous
import jax
import jax.numpy as jnp
import numpy as np
from jax.experimental import pallas as pl
from jax.experimental.pallas import tpu as pltpu

_B, _N, _D = 8, 2048, 256
_P = _D // 4
_TOPK = 8
_TR = 1024
_LANES = 128
_NCHUNK = _N // _LANES


def _graph_kernel(lt_ref, a_full_ref, a_tile_ref, wq_ref, bq_ref, wk_ref,
                  bk_ref, out_ref, k_scratch, thr_scratch):
    i = pl.program_id(1)

    @pl.when(i == 0)
    def _():
        a = a_full_ref[0]
        k = jax.lax.dot_general(a, wk_ref[...],
                                (((1,), (1,)), ((), ())),
                                preferred_element_type=jnp.float32)
        k_scratch[...] = k + bk_ref[...]

    temperature = jnp.clip(jnp.exp(lt_ref[0, 0]), 0.1, 10.0)
    inv_scale = 1.0 / (np.sqrt(float(_P)) * temperature)

    a_tile = a_tile_ref[0]
    q = jax.lax.dot_general(a_tile, wq_ref[...],
                            (((1,), (1,)), ((), ())),
                            preferred_element_type=jnp.float32)
    q = q + bq_ref[...]

    s = jax.lax.dot_general(q, k_scratch[...],
                            (((1,), (1,)), ((), ())),
                            preferred_element_type=jnp.float32)
    s = s * inv_scale

    chunks = [s[:, c * _LANES:(c + 1) * _LANES] for c in range(_NCHUNK)]
    t1 = chunks[0]
    t2 = jnp.full_like(t1, -jnp.inf)
    t3 = t2
    for c in chunks[1:]:
        lo = jnp.minimum(t1, c)
        t1 = jnp.maximum(t1, c)
        lo2 = jnp.minimum(t2, lo)
        t2 = jnp.maximum(t2, lo)
        t3 = jnp.maximum(t3, lo2)
    cand = jnp.concatenate([t1, t2, t3], axis=-1)

    m1 = jnp.max(cand, axis=-1, keepdims=True)
    w = jnp.where(cand == m1, -jnp.inf, cand)
    for _ in range(_TOPK - 2):
        m = jnp.max(w, axis=-1, keepdims=True)
        w = jnp.where(w == m, -jnp.inf, w)
    t8c = jnp.max(w, axis=-1, keepdims=True)

    cnt = jnp.sum((s >= t8c).astype(jnp.float32), axis=-1, keepdims=True)
    thr_scratch[...] = t8c

    @pl.when(jnp.any(cnt > _TOPK))
    def _():
        sw = jnp.where(s == m1, -jnp.inf, s)
        for _ in range(_TOPK - 2):
            mm = jnp.max(sw, axis=-1, keepdims=True)
            sw = jnp.where(sw == mm, -jnp.inf, sw)
        thr_full = jnp.max(sw, axis=-1, keepdims=True)
        thr_scratch[...] = jnp.where(cnt > _TOPK, thr_full, t8c)

    thr = thr_scratch[...]
    keep = s >= thr
    e = jnp.where(keep, jnp.exp(s - m1), 0.0)
    denom = jnp.sum(e, axis=-1, keepdims=True)
    out_ref[0] = e * (1.0 / denom)


def kernel(action_states, Wq, bq, Wk, bk, log_temperature):
    lt = jnp.reshape(log_temperature, (1, 1))
    bq2 = jnp.reshape(bq, (1, _P))
    bk2 = jnp.reshape(bk, (1, _P))

    return pl.pallas_call(
        _graph_kernel,
        grid=(_B, _N // _TR),
        in_specs=[
            pl.BlockSpec((1, 1), lambda b, i: (0, 0)),
            pl.BlockSpec((1, _N, _D), lambda b, i: (b, 0, 0)),
            pl.BlockSpec((1, _TR, _D), lambda b, i: (b, i, 0)),
            pl.BlockSpec((_P, _D), lambda b, i: (0, 0)),
            pl.BlockSpec((1, _P), lambda b, i: (0, 0)),
            pl.BlockSpec((_P, _D), lambda b, i: (0, 0)),
            pl.BlockSpec((1, _P), lambda b, i: (0, 0)),
        ],
        out_specs=pl.BlockSpec((1, _TR, _N), lambda b, i: (b, i, 0)),
        out_shape=jax.ShapeDtypeStruct((_B, _N, _N), jnp.float32),
        scratch_shapes=[pltpu.VMEM((_N, _P), jnp.float32),
                        pltpu.VMEM((_TR, 1), jnp.float32)],
        compiler_params=pltpu.CompilerParams(
            dimension_semantics=("parallel", "arbitrary")),
    )(lt, action_states, action_states, Wq, bq2, Wk, bk2)


# speedup vs baseline: 1.7691x; 1.0516x over previous
import jax
import jax.numpy as jnp
import numpy as np
from jax.experimental import pallas as pl
from jax.experimental.pallas import tpu as pltpu

_B, _N, _D = 8, 2048, 256
_P = _D // 4
_TOPK = 8
_TR = 1024
_LANES = 128
_NCHUNK = _N // _LANES


def _rowsum(x):
    ps = x[:, 0:_LANES]
    for c in range(1, x.shape[1] // _LANES):
        ps = ps + x[:, c * _LANES:(c + 1) * _LANES]
    return jnp.sum(ps, axis=-1, keepdims=True)


def _graph_kernel(lt_ref, a_full_ref, a_tile_ref, wq_ref, bq_ref, wk_ref,
                  bk_ref, out_ref, k_scratch):
    i = pl.program_id(1)

    @pl.when(i == 0)
    def _():
        a = a_full_ref[0]
        k = jax.lax.dot_general(a, wk_ref[...],
                                (((1,), (1,)), ((), ())),
                                preferred_element_type=jnp.float32)
        k_scratch[...] = k + bk_ref[...]

    temperature = jnp.clip(jnp.exp(lt_ref[0, 0]), 0.1, 10.0)
    c_scale = np.float32(np.log2(np.e)) / (np.sqrt(np.float32(_P)) * temperature)

    a_tile = a_tile_ref[0]
    q = jax.lax.dot_general(a_tile, wq_ref[...] * c_scale,
                            (((1,), (1,)), ((), ())),
                            preferred_element_type=jnp.float32)
    q = q + bq_ref[...] * c_scale

    s = jax.lax.dot_general(q, k_scratch[...],
                            (((1,), (1,)), ((), ())),
                            preferred_element_type=jnp.float32)

    chunks = [s[:, c * _LANES:(c + 1) * _LANES] for c in range(_NCHUNK)]
    t1 = chunks[0]
    t2 = jnp.full_like(t1, -jnp.inf)
    t3 = t2
    for c in chunks[1:]:
        lo = jnp.minimum(t1, c)
        t1 = jnp.maximum(t1, c)
        lo2 = jnp.minimum(t2, lo)
        t2 = jnp.maximum(t2, lo)
        t3 = jnp.maximum(t3, lo2)
    cand = jnp.concatenate([t1, t2, t3], axis=-1)

    m1 = jnp.max(cand, axis=-1, keepdims=True)
    w = jnp.where(cand == m1, -jnp.inf, cand)
    for _ in range(_TOPK - 2):
        m = jnp.max(w, axis=-1, keepdims=True)
        w = jnp.where(w == m, -jnp.inf, w)
    t8c = jnp.max(w, axis=-1, keepdims=True)

    keep0 = jnp.where(s >= t8c, 1.0, 0.0)
    cnt = _rowsum(keep0)
    e0 = keep0 * jnp.exp2(s - m1)
    denom0 = _rowsum(e0)
    out_ref[0] = e0 * (1.0 / denom0)

    @pl.when(jnp.any(cnt > _TOPK))
    def _():
        sw = jnp.where(s == m1, -jnp.inf, s)
        for _ in range(_TOPK - 2):
            mm = jnp.max(sw, axis=-1, keepdims=True)
            sw = jnp.where(sw == mm, -jnp.inf, sw)
        thr = jnp.where(cnt > _TOPK,
                        jnp.max(sw, axis=-1, keepdims=True), t8c)
        e = jnp.where(s >= thr, jnp.exp2(s - m1), 0.0)
        denom = jnp.sum(e, axis=-1, keepdims=True)
        out_ref[0] = e * (1.0 / denom)


def kernel(action_states, Wq, bq, Wk, bk, log_temperature):
    lt = jnp.reshape(log_temperature, (1, 1))
    bq2 = jnp.reshape(bq, (1, _P))
    bk2 = jnp.reshape(bk, (1, _P))

    return pl.pallas_call(
        _graph_kernel,
        grid=(_B, _N // _TR),
        in_specs=[
            pl.BlockSpec((1, 1), lambda b, i: (0, 0)),
            pl.BlockSpec((1, _N, _D), lambda b, i: (b, 0, 0)),
            pl.BlockSpec((1, _TR, _D), lambda b, i: (b, i, 0)),
            pl.BlockSpec((_P, _D), lambda b, i: (0, 0)),
            pl.BlockSpec((1, _P), lambda b, i: (0, 0)),
            pl.BlockSpec((_P, _D), lambda b, i: (0, 0)),
            pl.BlockSpec((1, _P), lambda b, i: (0, 0)),
        ],
        out_specs=pl.BlockSpec((1, _TR, _N), lambda b, i: (b, i, 0)),
        out_shape=jax.ShapeDtypeStruct((_B, _N, _N), jnp.float32),
        scratch_shapes=[pltpu.VMEM((_N, _P), jnp.float32)],
        compiler_params=pltpu.CompilerParams(
            dimension_semantics=("parallel", "arbitrary")),
    )(lt, action_states, action_states, Wq, bq2, Wk, bk2)
